# baseline (speedup 1.0000x reference)
import functools

import jax
import jax.numpy as jnp
from jax import lax
from jax.experimental import pallas as pl
from jax.experimental.pallas import tpu as pltpu
from jax.experimental.pallas import tpu_sc as plsc

NN = 100000
NP = 102400
EE = 6400000
EP = 6553600
NW = 32
EPW = EP // NW
BK = 2048
NB = EPW // BK
CH = 128
NCH = BK // CH
NPW = NP // NW
NLVL = 8


def _take16(x, idx):
    return lax.gather(
        x, idx[:, None],
        dimension_numbers=lax.GatherDimensionNumbers(
            offset_dims=(), collapsed_slice_dims=(0,), start_index_map=(0,)),
        slice_sizes=(1,),
        mode=lax.GatherScatterMode.PROMISE_IN_BOUNDS)


NB0 = 70
NB1 = 130


def _scatter_body(h_hbm, src_hbm, dst_hbm, neg_hbm, out_hbm,
                  agg, srcb0, dstb0, msgb0, srcb1, dstb1, msgb1, sem):
    cid = lax.axis_index("c")
    sid = lax.axis_index("s")
    wid = sid * 2 + cid
    base_blk = sid * (NB0 + NB1) + cid * NB0
    nblk = NB0 + cid * (NB1 - NB0)
    base = base_blk * BK
    pltpu.sync_copy(neg_hbm, agg)
    iota = lax.iota(jnp.int32, 16)
    bufs = ((srcb0, dstb0, msgb0), (srcb1, dstb1, msgb1))

    def _stage(b, p):
        sb, db, mb = bufs[p]
        off = base + b * BK
        pltpu.sync_copy(src_hbm.at[pl.ds(off, BK)], sb)
        pltpu.sync_copy(dst_hbm.at[pl.ds(off, BK)], db)
        for c in range(NCH):
            pltpu.async_copy(h_hbm.at[sb.at[pl.ds(c * CH, CH)]],
                             mb.at[pl.ds(c * CH, CH)], sem)

    def _wait(p):
        sb, db, mb = bufs[p]
        for c in range(NCH):
            pltpu.make_async_copy(h_hbm.at[sb.at[pl.ds(c * CH, CH)]],
                                  mb.at[pl.ds(c * CH, CH)], sem).wait()

    def _compute(p):
        db, mb = bufs[p][1], bufs[p][2]

        def vec(j, _):
            d = db[pl.ds(j * 16, 16)]
            m = mb[pl.ds(j * 16, 16)] + 1.0
            k, v = plsc.sort_key_val(d, m)
            for s in (1, 2, 4, 8):
                idx = jnp.maximum(iota - s, 0)
                ks = _take16(k, idx)
                vs = _take16(v, idx)
                v = jnp.where((iota >= s) & (ks == k), jnp.maximum(v, vs), v)
            kl = _take16(k, jnp.minimum(iota + 1, 15))
            last = (k != kl) | (iota == 15)
            old = plsc.load_gather(agg, [k])
            plsc.store_scatter(agg, [k], jnp.maximum(old, v), mask=last)
            return 0

        lax.fori_loop(0, BK // 16, vec, 0)

    _stage(0, 0)

    def pair(t, _):
        for phase in range(2):
            b = t * 2 + phase
            _wait(phase)

            @pl.when(b + 1 < nblk)
            def _():
                _stage(b + 1, 1 - phase)

            _compute(phase)
        return 0

    lax.fori_loop(0, nblk // 2, pair, 0)
    pltpu.sync_copy(agg, out_hbm.at[wid])


def _apply_body(h_hbm, aggs_hbm, lvl_hbm, ivec_hbm, out_hbm,
                hbuf, lbuf, acc, tmp, ivec, sem):
    wid = lax.axis_index("s") * 2 + lax.axis_index("c")
    base = wid * NPW
    pltpu.sync_copy(h_hbm.at[pl.ds(base, NPW)], hbuf)
    pltpu.sync_copy(lvl_hbm.at[pl.ds(base, NPW)], lbuf)
    pltpu.sync_copy(ivec_hbm, ivec)
    pltpu.sync_copy(aggs_hbm.at[0, pl.ds(base, NPW)], acc)
    for s in range(1, NW):
        pltpu.sync_copy(aggs_hbm.at[s, pl.ds(base, NPW)], tmp)

        def mx(j, _):
            sl = pl.ds(j * 16, 16)
            acc[sl] = jnp.maximum(acc[sl], tmp[sl])
            return 0

        lax.fori_loop(0, NPW // 16, mx, 0)
    iv = ivec[...]

    def sel(j, _):
        sl = pl.ds(j * 16, 16)
        hbuf[sl] = jnp.where(lbuf[sl] == iv, acc[sl], hbuf[sl])
        return 0

    lax.fori_loop(0, NPW // 16, sel, 0)
    pltpu.sync_copy(hbuf, out_hbm.at[pl.ds(base, NPW)])


_MESH = plsc.VectorSubcoreMesh(core_axis_name="c", subcore_axis_name="s")
_CPARAMS = pltpu.CompilerParams(needs_layout_passes=False)

_scatter = functools.partial(
    pl.kernel,
    out_type=jax.ShapeDtypeStruct((NW, NP), jnp.float32),
    mesh=_MESH,
    compiler_params=_CPARAMS,
    scratch_types=[
        pltpu.VMEM((NP,), jnp.float32),
        pltpu.VMEM((BK,), jnp.int32),
        pltpu.VMEM((BK,), jnp.int32),
        pltpu.VMEM((BK,), jnp.float32),
        pltpu.VMEM((BK,), jnp.int32),
        pltpu.VMEM((BK,), jnp.int32),
        pltpu.VMEM((BK,), jnp.float32),
        pltpu.SemaphoreType.DMA,
    ],
)(_scatter_body)

_apply = functools.partial(
    pl.kernel,
    out_type=jax.ShapeDtypeStruct((NP,), jnp.float32),
    mesh=_MESH,
    compiler_params=_CPARAMS,
    scratch_types=[
        pltpu.VMEM((NPW,), jnp.float32),
        pltpu.VMEM((NPW,), jnp.int32),
        pltpu.VMEM((NPW,), jnp.float32),
        pltpu.VMEM((NPW,), jnp.float32),
        pltpu.VMEM((16,), jnp.int32),
        pltpu.SemaphoreType.DMA,
    ],
)(_apply_body)


def kernel(hdr, edge_index, node_level):
    src = edge_index[0]
    dst = edge_index[1]
    h = jnp.concatenate([hdr, jnp.zeros((NP - NN,), jnp.float32)])
    lvl = jnp.concatenate([node_level, jnp.zeros((NP - NN,), jnp.int32)])
    srcp = jnp.concatenate([src, jnp.zeros((EP - EE,), jnp.int32)])
    dstp = jnp.concatenate([dst, jnp.full((EP - EE,), NP - 1, jnp.int32)])
    neg = jnp.full((NP,), -jnp.inf, jnp.float32)
    for i in range(1, NLVL):
        aggs = _scatter(h, srcp, dstp, neg)
        h = _apply(h, aggs, lvl, jnp.full((16,), i, jnp.int32))
    return h[:NN]


# speedup vs baseline: 1.2159x; 1.2159x over previous
import functools

import jax
import jax.numpy as jnp
from jax import lax
from jax.experimental import pallas as pl
from jax.experimental.pallas import tpu as pltpu
from jax.experimental.pallas import tpu_sc as plsc

NN = 100000
NP = 102400
EE = 6400000
EP = 6553600
NW = 32
EPW = EP // NW
BK = 2048
NB = EPW // BK
CH = 128
NCH = BK // CH
NPW = NP // NW
NLVL = 8


def _take16(x, idx):
    return lax.gather(
        x, idx[:, None],
        dimension_numbers=lax.GatherDimensionNumbers(
            offset_dims=(), collapsed_slice_dims=(0,), start_index_map=(0,)),
        slice_sizes=(1,),
        mode=lax.GatherScatterMode.PROMISE_IN_BOUNDS)


NB0 = 130
NB1 = 70


def _scatter_body(h_hbm, src_hbm, dst_hbm, neg_hbm, out_hbm,
                  agg, srcb0, dstb0, msgb0, srcb1, dstb1, msgb1, sem):
    cid = lax.axis_index("c")
    sid = lax.axis_index("s")
    wid = sid * 2 + cid
    base_blk = sid * (NB0 + NB1) + cid * NB0
    nblk = NB0 + cid * (NB1 - NB0)
    base = base_blk * BK
    pltpu.sync_copy(neg_hbm, agg)
    iota = lax.iota(jnp.int32, 16)
    bufs = ((srcb0, dstb0, msgb0), (srcb1, dstb1, msgb1))

    def _stage(b, p):
        sb, db, mb = bufs[p]
        off = base + b * BK
        pltpu.sync_copy(src_hbm.at[pl.ds(off, BK)], sb)
        pltpu.sync_copy(dst_hbm.at[pl.ds(off, BK)], db)
        for c in range(NCH):
            pltpu.async_copy(h_hbm.at[sb.at[pl.ds(c * CH, CH)]],
                             mb.at[pl.ds(c * CH, CH)], sem)

    def _wait(p):
        sb, db, mb = bufs[p]
        for c in range(NCH):
            pltpu.make_async_copy(h_hbm.at[sb.at[pl.ds(c * CH, CH)]],
                                  mb.at[pl.ds(c * CH, CH)], sem).wait()

    def _compute(p):
        db, mb = bufs[p][1], bufs[p][2]

        def vec(j, _):
            d = db[pl.ds(j * 16, 16)]
            m = mb[pl.ds(j * 16, 16)] + 1.0
            k, v = plsc.sort_key_val(d, m)
            for s in (1, 2, 4, 8):
                idx = jnp.maximum(iota - s, 0)
                ks = _take16(k, idx)
                vs = _take16(v, idx)
                v = jnp.where((iota >= s) & (ks == k), jnp.maximum(v, vs), v)
            kl = _take16(k, jnp.minimum(iota + 1, 15))
            last = (k != kl) | (iota == 15)
            old = plsc.load_gather(agg, [k])
            plsc.store_scatter(agg, [k], jnp.maximum(old, v), mask=last)
            return 0

        lax.fori_loop(0, BK // 16, vec, 0)

    _stage(0, 0)

    def pair(t, _):
        for phase in range(2):
            b = t * 2 + phase
            _wait(phase)

            @pl.when(b + 1 < nblk)
            def _():
                _stage(b + 1, 1 - phase)

            _compute(phase)
        return 0

    lax.fori_loop(0, nblk // 2, pair, 0)
    pltpu.sync_copy(agg, out_hbm.at[wid])


def _apply_body(h_hbm, aggs_hbm, lvl_hbm, ivec_hbm, out_hbm,
                hbuf, lbuf, acc, tmp, ivec, sem):
    wid = lax.axis_index("s") * 2 + lax.axis_index("c")
    base = wid * NPW
    pltpu.sync_copy(h_hbm.at[pl.ds(base, NPW)], hbuf)
    pltpu.sync_copy(lvl_hbm.at[pl.ds(base, NPW)], lbuf)
    pltpu.sync_copy(ivec_hbm, ivec)
    pltpu.sync_copy(aggs_hbm.at[0, pl.ds(base, NPW)], acc)
    for s in range(1, NW):
        pltpu.sync_copy(aggs_hbm.at[s, pl.ds(base, NPW)], tmp)

        def mx(j, _):
            sl = pl.ds(j * 16, 16)
            acc[sl] = jnp.maximum(acc[sl], tmp[sl])
            return 0

        lax.fori_loop(0, NPW // 16, mx, 0)
    iv = ivec[...]

    def sel(j, _):
        sl = pl.ds(j * 16, 16)
        hbuf[sl] = jnp.where(lbuf[sl] == iv, acc[sl], hbuf[sl])
        return 0

    lax.fori_loop(0, NPW // 16, sel, 0)
    pltpu.sync_copy(hbuf, out_hbm.at[pl.ds(base, NPW)])


_MESH = plsc.VectorSubcoreMesh(core_axis_name="c", subcore_axis_name="s")
_CPARAMS = pltpu.CompilerParams(needs_layout_passes=False)

_scatter = functools.partial(
    pl.kernel,
    out_type=jax.ShapeDtypeStruct((NW, NP), jnp.float32),
    mesh=_MESH,
    compiler_params=_CPARAMS,
    scratch_types=[
        pltpu.VMEM((NP,), jnp.float32),
        pltpu.VMEM((BK,), jnp.int32),
        pltpu.VMEM((BK,), jnp.int32),
        pltpu.VMEM((BK,), jnp.float32),
        pltpu.VMEM((BK,), jnp.int32),
        pltpu.VMEM((BK,), jnp.int32),
        pltpu.VMEM((BK,), jnp.float32),
        pltpu.SemaphoreType.DMA,
    ],
)(_scatter_body)

_apply = functools.partial(
    pl.kernel,
    out_type=jax.ShapeDtypeStruct((NP,), jnp.float32),
    mesh=_MESH,
    compiler_params=_CPARAMS,
    scratch_types=[
        pltpu.VMEM((NPW,), jnp.float32),
        pltpu.VMEM((NPW,), jnp.int32),
        pltpu.VMEM((NPW,), jnp.float32),
        pltpu.VMEM((NPW,), jnp.float32),
        pltpu.VMEM((16,), jnp.int32),
        pltpu.SemaphoreType.DMA,
    ],
)(_apply_body)


def kernel(hdr, edge_index, node_level):
    src = edge_index[0]
    dst = edge_index[1]
    h = jnp.concatenate([hdr, jnp.zeros((NP - NN,), jnp.float32)])
    lvl = jnp.concatenate([node_level, jnp.zeros((NP - NN,), jnp.int32)])
    srcp = jnp.concatenate([src, jnp.zeros((EP - EE,), jnp.int32)])
    dstp = jnp.concatenate([dst, jnp.full((EP - EE,), NP - 1, jnp.int32)])
    neg = jnp.full((NP,), -jnp.inf, jnp.float32)
    for i in range(1, NLVL):
        aggs = _scatter(h, srcp, dstp, neg)
        h = _apply(h, aggs, lvl, jnp.full((16,), i, jnp.int32))
    return h[:NN]


# speedup vs baseline: 2.2571x; 1.8564x over previous
import functools

import jax
import jax.numpy as jnp
from jax import lax
from jax.experimental import pallas as pl
from jax.experimental.pallas import tpu as pltpu
from jax.experimental.pallas import tpu_sc as plsc

NN = 100000
NP = 102400
EE = 6400000
EP = 6553600
NW = 32
EPW = EP // NW
BK = 2048
NB = EPW // BK
CH = 128
NCH = BK // CH
NPW = NP // NW
NLVL = 8


def _take16(x, idx):
    return lax.gather(
        x, idx[:, None],
        dimension_numbers=lax.GatherDimensionNumbers(
            offset_dims=(), collapsed_slice_dims=(0,), start_index_map=(0,)),
        slice_sizes=(1,),
        mode=lax.GatherScatterMode.PROMISE_IN_BOUNDS)


NB0 = 100
NB1 = 100


def _scatter_body(h_hbm, src_hbm, dst_hbm, neg_hbm, out_hbm,
                  agg, srcb0, dstb0, msgb0, srcb1, dstb1, msgb1, hsh, sem):
    cid = lax.axis_index("c")
    sid = lax.axis_index("s")
    wid = sid * 2 + cid
    base_blk = sid * (NB0 + NB1) + cid * NB0
    nblk = NB0 + cid * (NB1 - NB0)
    base = base_blk * BK

    @pl.when(sid == 0)
    def _():
        pltpu.sync_copy(h_hbm, hsh)

    pltpu.sync_copy(neg_hbm, agg)
    plsc.subcore_barrier()
    iota = lax.iota(jnp.int32, 16)
    bufs = ((srcb0, dstb0, msgb0), (srcb1, dstb1, msgb1))

    def _stage(b, p):
        sb, db, mb = bufs[p]
        off = base + b * BK
        pltpu.sync_copy(src_hbm.at[pl.ds(off, BK)], sb)
        pltpu.sync_copy(dst_hbm.at[pl.ds(off, BK)], db)
        for c in range(NCH):
            pltpu.async_copy(hsh.at[sb.at[pl.ds(c * CH, CH)]],
                             mb.at[pl.ds(c * CH, CH)], sem)

    def _wait(p):
        sb, db, mb = bufs[p]
        for c in range(NCH):
            pltpu.make_async_copy(hsh.at[sb.at[pl.ds(c * CH, CH)]],
                                  mb.at[pl.ds(c * CH, CH)], sem).wait()

    def _compute(p):
        db, mb = bufs[p][1], bufs[p][2]

        def vec(j, _):
            d = db[pl.ds(j * 16, 16)]
            m = mb[pl.ds(j * 16, 16)] + 1.0
            k, v = plsc.sort_key_val(d, m)
            for s in (1, 2, 4, 8):
                idx = jnp.maximum(iota - s, 0)
                ks = _take16(k, idx)
                vs = _take16(v, idx)
                v = jnp.where((iota >= s) & (ks == k), jnp.maximum(v, vs), v)
            kl = _take16(k, jnp.minimum(iota + 1, 15))
            last = (k != kl) | (iota == 15)
            old = plsc.load_gather(agg, [k])
            plsc.store_scatter(agg, [k], jnp.maximum(old, v), mask=last)
            return 0

        lax.fori_loop(0, BK // 16, vec, 0)

    _stage(0, 0)

    def pair(t, _):
        for phase in range(2):
            b = t * 2 + phase
            _wait(phase)

            @pl.when(b + 1 < nblk)
            def _():
                _stage(b + 1, 1 - phase)

            _compute(phase)
        return 0

    lax.fori_loop(0, nblk // 2, pair, 0)
    pltpu.sync_copy(agg, out_hbm.at[wid])


def _apply_body(h_hbm, aggs_hbm, lvl_hbm, ivec_hbm, out_hbm,
                hbuf, lbuf, acc, tmp, ivec, sem):
    wid = lax.axis_index("s") * 2 + lax.axis_index("c")
    base = wid * NPW
    pltpu.sync_copy(h_hbm.at[pl.ds(base, NPW)], hbuf)
    pltpu.sync_copy(lvl_hbm.at[pl.ds(base, NPW)], lbuf)
    pltpu.sync_copy(ivec_hbm, ivec)
    pltpu.sync_copy(aggs_hbm.at[0, pl.ds(base, NPW)], acc)
    for s in range(1, NW):
        pltpu.sync_copy(aggs_hbm.at[s, pl.ds(base, NPW)], tmp)

        def mx(j, _):
            sl = pl.ds(j * 16, 16)
            acc[sl] = jnp.maximum(acc[sl], tmp[sl])
            return 0

        lax.fori_loop(0, NPW // 16, mx, 0)
    iv = ivec[...]

    def sel(j, _):
        sl = pl.ds(j * 16, 16)
        hbuf[sl] = jnp.where(lbuf[sl] == iv, acc[sl], hbuf[sl])
        return 0

    lax.fori_loop(0, NPW // 16, sel, 0)
    pltpu.sync_copy(hbuf, out_hbm.at[pl.ds(base, NPW)])


_MESH = plsc.VectorSubcoreMesh(core_axis_name="c", subcore_axis_name="s")
_CPARAMS = pltpu.CompilerParams(needs_layout_passes=False)

_scatter = functools.partial(
    pl.kernel,
    out_type=jax.ShapeDtypeStruct((NW, NP), jnp.float32),
    mesh=_MESH,
    compiler_params=_CPARAMS,
    scratch_types=[
        pltpu.VMEM((NP,), jnp.float32),
        pltpu.VMEM((BK,), jnp.int32),
        pltpu.VMEM((BK,), jnp.int32),
        pltpu.VMEM((BK,), jnp.float32),
        pltpu.VMEM((BK,), jnp.int32),
        pltpu.VMEM((BK,), jnp.int32),
        pltpu.VMEM((BK,), jnp.float32),
        pltpu.VMEM_SHARED((NP,), jnp.float32),
        pltpu.SemaphoreType.DMA,
    ],
)(_scatter_body)

_apply = functools.partial(
    pl.kernel,
    out_type=jax.ShapeDtypeStruct((NP,), jnp.float32),
    mesh=_MESH,
    compiler_params=_CPARAMS,
    scratch_types=[
        pltpu.VMEM((NPW,), jnp.float32),
        pltpu.VMEM((NPW,), jnp.int32),
        pltpu.VMEM((NPW,), jnp.float32),
        pltpu.VMEM((NPW,), jnp.float32),
        pltpu.VMEM((16,), jnp.int32),
        pltpu.SemaphoreType.DMA,
    ],
)(_apply_body)


def kernel(hdr, edge_index, node_level):
    src = edge_index[0]
    dst = edge_index[1]
    h = jnp.concatenate([hdr, jnp.zeros((NP - NN,), jnp.float32)])
    lvl = jnp.concatenate([node_level, jnp.zeros((NP - NN,), jnp.int32)])
    srcp = jnp.concatenate([src, jnp.zeros((EP - EE,), jnp.int32)])
    dstp = jnp.concatenate([dst, jnp.full((EP - EE,), NP - 1, jnp.int32)])
    neg = jnp.full((NP,), -jnp.inf, jnp.float32)
    for i in range(1, NLVL):
        aggs = _scatter(h, srcp, dstp, neg)
        h = _apply(h, aggs, lvl, jnp.full((16,), i, jnp.int32))
    return h[:NN]
